# baseline (speedup 1.0000x reference)
import functools

import jax
import jax.numpy as jnp
from jax import lax
from jax.experimental import pallas as pl
from jax.experimental.pallas import tpu as pltpu
from jax.experimental.pallas import tpu_sc as plsc

_D = 256
_K = 8192
_TB = 512
_KB = 1024
_TB2 = 2048
_CHUNK = 128


def _argmin_body(z_ref, e_ref, idx_ref, min_s, arg_s):
    k = pl.program_id(1)
    nk = pl.num_programs(1)

    @pl.when(k == 0)
    def _init():
        min_s[...] = jnp.full_like(min_s, jnp.inf)
        arg_s[...] = jnp.zeros_like(arg_s)

    z = z_ref[...]
    e = e_ref[...]
    zs = jnp.sum(z * z, axis=1, keepdims=True)
    es = jnp.sum(e * e, axis=1)
    prod = lax.dot_general(z, e, (((1,), (1,)), ((), ())),
                           preferred_element_type=jnp.float32,
                           precision=lax.Precision.DEFAULT)
    s = zs - 2.0 * prod + es[None, :]
    local_min = jnp.min(s, axis=1)
    cols = lax.broadcasted_iota(jnp.int32, s.shape, 1)
    local_arg = jnp.min(jnp.where(s == local_min[:, None], cols, _KB), axis=1)
    better = local_min < min_s[...]
    arg_s[...] = jnp.where(better, k * _KB + local_arg, arg_s[...])
    min_s[...] = jnp.where(better, local_min, min_s[...])

    @pl.when(k == nk - 1)
    def _out():
        idx_ref[...] = arg_s[...]


def _argmin_indices(z, ew):
    n = z.shape[0]
    return pl.pallas_call(
        _argmin_body,
        grid=(n // _TB, _K // _KB),
        in_specs=[
            pl.BlockSpec((_TB, _D), lambda t, k: (t, 0)),
            pl.BlockSpec((_KB, _D), lambda t, k: (k, 0)),
        ],
        out_specs=pl.BlockSpec((_TB,), lambda t, k: (t,)),
        out_shape=jax.ShapeDtypeStruct((n,), jnp.int32),
        scratch_shapes=[pltpu.VMEM((_TB,), jnp.float32),
                        pltpu.VMEM((_TB,), jnp.int32)],
    )(z, ew)


def _sc_gather(ew, idx2):
    info = plsc.get_sparse_core_info()
    nc, ns = info.num_cores, info.num_subcores
    nw = nc * ns
    n = idx2.shape[0] * idx2.shape[1]
    bpw = n // nw
    cpw = bpw // _CHUNK
    mesh = plsc.VectorSubcoreMesh(core_axis_name="c", subcore_axis_name="s")

    @functools.partial(
        pl.kernel, mesh=mesh,
        out_type=jax.ShapeDtypeStruct((n, _D), jnp.float32),
        scratch_types=[
            pltpu.VMEM((cpw, _CHUNK), jnp.int32),
            pltpu.VMEM((bpw, _D), jnp.float32),
            pltpu.SemaphoreType.DMA,
        ],
    )
    def gather_kernel(table_hbm, idx_hbm, out_hbm, idx_v, rows_v, sem):
        wid = lax.axis_index("s") * nc + lax.axis_index("c")
        pltpu.sync_copy(idx_hbm.at[pl.ds(wid * cpw, cpw)], idx_v)
        for j in range(cpw):
            pltpu.async_copy(table_hbm.at[idx_v.at[j]],
                             rows_v.at[pl.ds(j * _CHUNK, _CHUNK)], sem).wait()
        pltpu.sync_copy(rows_v, out_hbm.at[pl.ds(wid * bpw, bpw)])

    return gather_kernel(ew, idx2)


def _recon_body(z_ref, q_ref, r_ref, loss_ref, acc):
    t = pl.program_id(0)
    nt = pl.num_programs(0)

    @pl.when(t == 0)
    def _init():
        acc[0] = 0.0

    z = z_ref[...]
    d = q_ref[...] - z
    r_ref[...] = z + d
    acc[0] += jnp.sum(d * d)

    @pl.when(t == nt - 1)
    def _out():
        loss_ref[0] = acc[0] * (1.25 / float(z_ref.shape[0] * _D * nt))


def _recon_and_loss(z, q):
    n = z.shape[0]
    return pl.pallas_call(
        _recon_body,
        grid=(n // _TB2,),
        in_specs=[
            pl.BlockSpec((_TB2, _D), lambda t: (t, 0)),
            pl.BlockSpec((_TB2, _D), lambda t: (t, 0)),
        ],
        out_specs=[
            pl.BlockSpec((_TB2, _D), lambda t: (t, 0)),
            pl.BlockSpec(memory_space=pltpu.SMEM),
        ],
        out_shape=[jax.ShapeDtypeStruct((n, _D), jnp.float32),
                   jax.ShapeDtypeStruct((1,), jnp.float32)],
        scratch_shapes=[pltpu.SMEM((1,), jnp.float32)],
    )(z, q)


def kernel(x, embedding_weight):
    b, d, h, w = x.shape
    z = jnp.transpose(x, (0, 2, 3, 1)).reshape(-1, d)
    idx = _argmin_indices(z, embedding_weight)
    q = _sc_gather(embedding_weight, idx.reshape(-1, _CHUNK))
    recon, loss = _recon_and_loss(z, q)
    x_recon = jnp.transpose(recon.reshape(b, h, w, d), (0, 3, 1, 2))
    return x_recon, loss.reshape(())
